# baseline (speedup 1.0000x reference)
import dataclasses
import functools

import jax
import jax.numpy as jnp
from jax import lax
from jax.experimental import pallas as pl
from jax.experimental.pallas import tpu as pltpu
from jax.experimental.pallas import tpu_sc as plsc

LANES = 16
WINDOW = 128


def kernel(input_BC, table):
    B, C = input_BC.shape
    V = table.shape[1]
    N = B * C

    idx_flat = input_BC.reshape(1, N).astype(jnp.int32)
    pos_flat = jnp.broadcast_to(
        jnp.arange(C, dtype=jnp.float32), (B, C)
    ).reshape(1, N)

    mesh = plsc.VectorSubcoreMesh(core_axis_name="c", subcore_axis_name="s")

    cp = pltpu.CompilerParams(
        needs_layout_passes=False, use_tc_tiling_on_sc=False
    )

    @functools.partial(
        pl.kernel,
        out_type=jax.ShapeDtypeStruct((N, V), jnp.float32),
        mesh=mesh,
        compiler_params=cp,
    )
    def embed_lookup(table_hbm, idx_hbm, pos_hbm, out_hbm):
        def body(i_vmem, p_vmem, o_vmem):
            pltpu.sync_copy(table_hbm.at[i_vmem.at[0]], o_vmem)
            for k in range(WINDOW // LANES):
                rows = lax.iota(jnp.int32, LANES) + (k * LANES)
                cols = jnp.full((LANES,), V - 1, jnp.int32)
                pv = p_vmem[0, pl.ds(k * LANES, LANES)]
                plsc.store_scatter(o_vmem, [rows, cols], pv)

        pltpu.emit_pipeline(
            body,
            grid=(N // WINDOW,),
            in_specs=[
                pl.BlockSpec((1, WINDOW), lambda i: (0, i)),
                pl.BlockSpec((1, WINDOW), lambda i: (0, i)),
            ],
            out_specs=[pl.BlockSpec((WINDOW, V), lambda i: (i, 0))],
            core_axis_name=("c", "s"),
            dimension_semantics=(pltpu.PARALLEL,),
        )(idx_hbm, pos_hbm, out_hbm)

    out = embed_lookup(table, idx_flat, pos_flat)
    return out.reshape(B, C, V)


# speedup vs baseline: 1.0917x; 1.0917x over previous
import functools

import jax
import jax.numpy as jnp
from jax import lax
from jax.experimental import pallas as pl
from jax.experimental.pallas import tpu as pltpu
from jax.experimental.pallas import tpu_sc as plsc

LANES = 16
CHUNK = 128
NBUF = 8
LEAD = 4
NW = 32


def kernel(input_BC, table):
    B, C = input_BC.shape
    V = table.shape[1]
    N = B * C
    per_w = N // NW
    n_chunks = per_w // CHUNK

    idx_flat = input_BC.reshape(N).astype(jnp.int32)

    mesh = plsc.VectorSubcoreMesh(core_axis_name="c", subcore_axis_name="s")
    cp = pltpu.CompilerParams(
        needs_layout_passes=False, use_tc_tiling_on_sc=False
    )

    scratch = (
        [pltpu.VMEM((per_w,), jnp.int32)]
        + [pltpu.VMEM((CHUNK, V), jnp.float32) for _ in range(NBUF)]
        + [pltpu.SemaphoreType.DMA for _ in range(2 * NBUF + 1)]
    )

    @functools.partial(
        pl.kernel,
        out_type=jax.ShapeDtypeStruct((N, V), jnp.float32),
        mesh=mesh,
        compiler_params=cp,
        scratch_types=scratch,
    )
    def embed(table_hbm, idx_hbm, out_hbm, idx_v, *rest):
        bufs = rest[:NBUF]
        gsem = rest[NBUF:2 * NBUF]
        ssem = rest[2 * NBUF:3 * NBUF]
        isem = rest[3 * NBUF]

        wid = lax.axis_index("s") * 2 + lax.axis_index("c")
        base = wid * per_w

        pltpu.make_async_copy(
            idx_hbm.at[pl.ds(base, per_w)], idx_v, isem
        ).start()
        pltpu.make_async_copy(
            idx_hbm.at[pl.ds(base, per_w)], idx_v, isem
        ).wait()

        def gather_desc(chunk, b):
            return pltpu.make_async_copy(
                table_hbm.at[idx_v.at[pl.ds(chunk * CHUNK, CHUNK)]],
                bufs[b],
                gsem[b],
            )

        def store_desc(chunk, b):
            return pltpu.make_async_copy(
                bufs[b],
                out_hbm.at[pl.ds(base + chunk * CHUNK, CHUNK)],
                ssem[b],
            )

        for c in range(LEAD):
            gather_desc(c, c % NBUF).start()

        lane = lax.iota(jnp.int32, LANES)
        cols = jnp.full((LANES,), V - 1, jnp.int32)
        cmod = jnp.full((LANES,), C, jnp.int32)

        @pl.loop(0, n_chunks, step=NBUF)
        def _(g0):
            for b in range(NBUF):
                g = g0 + b
                gc = g + LEAD
                bc = (b + LEAD) % NBUF

                @pl.when(jnp.logical_and(gc < n_chunks, gc >= NBUF))
                def _():
                    store_desc(gc - NBUF, bc).wait()
                    gather_desc(gc, bc).start()

                @pl.when(jnp.logical_and(gc < n_chunks, gc < NBUF))
                def _():
                    gather_desc(gc, bc).start()

                gather_desc(g, b).wait()
                row0 = base + g * CHUNK
                for k in range(CHUNK // LANES):
                    rows = lane + (k * LANES)
                    pos = lax.rem(lane + (row0 + k * LANES), cmod)
                    plsc.store_scatter(
                        bufs[b], [rows, cols], pos.astype(jnp.float32)
                    )
                store_desc(g, b).start()

        for b in range(NBUF):
            store_desc(n_chunks - NBUF + b, b).wait()

    out = embed(table, idx_flat)
    return out.reshape(B, C, V)
